# baseline (speedup 1.0000x reference)
import jax
import jax.numpy as jnp
from jax import lax
from jax.experimental import pallas as pl
from jax.experimental.pallas import tpu as pltpu
from jax.experimental.pallas import tpu_sc as plsc

NC, NS = 2, 16
NW = NC * NS
CHUNK = 16
NBUF = 2


def _sc_body(table_hbm, out_hbm, buf, rsem, wsem):
    wid = lax.axis_index("s") * NC + lax.axis_index("c")
    rows_per_w = table_hbm.shape[0] // NW
    nchunk = rows_per_w // CHUNK
    base = wid * rows_per_w

    def rd(chunk, slot):
        return pltpu.make_async_copy(
            table_hbm.at[pl.ds(base + chunk * CHUNK, CHUNK)],
            buf.at[slot],
            rsem.at[slot],
        )

    def wr(b, chunk, slot):
        return pltpu.make_async_copy(
            buf.at[slot],
            out_hbm.at[b, pl.ds(base + chunk * CHUNK, CHUNK)],
            wsem.at[slot],
        )

    for s in range(NBUF):
        rd(s, s).start()

    def outer(g, carry):
        for s in range(NBUF):
            chunk = g * NBUF + s
            rd(chunk, s).wait()
            for b in range(4):
                wr(b, chunk, s).start()
        for s in range(NBUF):
            chunk = g * NBUF + s
            for b in range(4):
                wr(b, chunk, s).wait()
            nxt = chunk + NBUF

            @pl.when(nxt < nchunk)
            def _():
                rd(nxt, s).start()

        return carry

    lax.fori_loop(0, nchunk // NBUF, outer, 0)


def kernel(B, T, pos_weight):
    t_static, d = pos_weight.shape
    run = pl.kernel(
        _sc_body,
        out_type=jax.ShapeDtypeStruct((4, t_static, d), pos_weight.dtype),
        mesh=plsc.VectorSubcoreMesh(core_axis_name="c", subcore_axis_name="s"),
        scratch_types=[
            pltpu.VMEM((NBUF, CHUNK, d), jnp.float32),
            pltpu.SemaphoreType.DMA((NBUF,)),
            pltpu.SemaphoreType.DMA((NBUF,)),
        ],
    )
    return run(pos_weight)


# speedup vs baseline: 1.0560x; 1.0560x over previous
import jax
import jax.numpy as jnp
from jax import lax
from jax.experimental import pallas as pl
from jax.experimental.pallas import tpu as pltpu
from jax.experimental.pallas import tpu_sc as plsc

NC, NS = 2, 16
NW = NC * NS
CHUNK = 32


def _sc_body(table_hbm, out_hbm, buf, wsem):
    wid = lax.axis_index("s") * NC + lax.axis_index("c")
    rows_per_w = table_hbm.shape[0] // NW
    nchunk = rows_per_w // CHUNK
    base = wid * rows_per_w

    def wr(b, chunk):
        return pltpu.make_async_copy(
            buf,
            out_hbm.at[b, pl.ds(base + chunk * CHUNK, CHUNK)],
            wsem,
        )

    def step(i, carry):
        pltpu.sync_copy(table_hbm.at[pl.ds(base + i * CHUNK, CHUNK)], buf)
        for b in range(4):
            wr(b, i).start()
        for b in range(4):
            wr(b, i).wait()
        return carry

    lax.fori_loop(0, nchunk, step, 0)


def kernel(B, T, pos_weight):
    t_static, d = pos_weight.shape
    run = pl.kernel(
        _sc_body,
        out_type=jax.ShapeDtypeStruct((4, t_static, d), pos_weight.dtype),
        mesh=plsc.VectorSubcoreMesh(core_axis_name="c", subcore_axis_name="s"),
        scratch_types=[
            pltpu.VMEM((CHUNK, d), jnp.float32),
            pltpu.SemaphoreType.DMA,
        ],
    )
    return run(pos_weight)


# speedup vs baseline: 1.3798x; 1.3066x over previous
import jax
import jax.numpy as jnp
from jax import lax
from jax.experimental import pallas as pl
from jax.experimental.pallas import tpu as pltpu

T_BLK = 256
NBUF = 8


def _dma_body(w_hbm, o_hbm, buf, rsem, wsem):
    n = w_hbm.shape[0] // T_BLK

    def rd(i, s):
        return pltpu.make_async_copy(
            w_hbm.at[pl.ds(i * T_BLK, T_BLK)], buf.at[s], rsem.at[s]
        )

    def wr(b, i, s):
        return pltpu.make_async_copy(
            buf.at[s], o_hbm.at[b, pl.ds(i * T_BLK, T_BLK)], wsem.at[s]
        )

    for s in range(NBUF):
        rd(s, s).start()

    def step(g, carry):
        for s in range(NBUF):
            i = g * NBUF + s
            rd(i, s).wait()
            for b in range(4):
                wr(b, i, s).start()
        for s in range(NBUF):
            i = g * NBUF + s
            for b in range(4):
                wr(b, i, s).wait()
            nxt = i + NBUF

            @pl.when(nxt < n)
            def _():
                rd(nxt, s).start()

        return carry

    lax.fori_loop(0, n // NBUF, step, 0)


def kernel(B, T, pos_weight):
    t_static, d = pos_weight.shape
    out = pl.pallas_call(
        _dma_body,
        in_specs=[pl.BlockSpec(memory_space=pltpu.MemorySpace.HBM)],
        out_specs=pl.BlockSpec(memory_space=pltpu.MemorySpace.HBM),
        out_shape=jax.ShapeDtypeStruct((4, t_static, d), pos_weight.dtype),
        scratch_shapes=[
            pltpu.VMEM((NBUF, T_BLK, d), jnp.float32),
            pltpu.SemaphoreType.DMA((NBUF,)),
            pltpu.SemaphoreType.DMA((NBUF,)),
        ],
    )(pos_weight)
    return out


# speedup vs baseline: 1.4222x; 1.0308x over previous
import jax
import jax.numpy as jnp
from jax import lax
from jax.experimental import pallas as pl
from jax.experimental.pallas import tpu as pltpu

T_BLK = 512
NBUF = 8


def _dma_body(w_hbm, o_hbm, buf, rsem, wsem):
    n = w_hbm.shape[0] // T_BLK

    def rd(i, s):
        return pltpu.make_async_copy(
            w_hbm.at[pl.ds(i * T_BLK, T_BLK)], buf.at[s], rsem.at[s]
        )

    def wr(b, i, s):
        return pltpu.make_async_copy(
            buf.at[s], o_hbm.at[b, pl.ds(i * T_BLK, T_BLK)], wsem.at[s]
        )

    for s in range(NBUF):
        rd(s, s).start()

    def step(g, carry):
        for s in range(NBUF):
            i = g * NBUF + s
            rd(i, s).wait()
            for b in range(4):
                wr(b, i, s).start()
        for s in range(NBUF):
            i = g * NBUF + s
            for b in range(4):
                wr(b, i, s).wait()
            nxt = i + NBUF

            @pl.when(nxt < n)
            def _():
                rd(nxt, s).start()

        return carry

    lax.fori_loop(0, n // NBUF, step, 0)


def kernel(B, T, pos_weight):
    t_static, d = pos_weight.shape
    out = pl.pallas_call(
        _dma_body,
        in_specs=[pl.BlockSpec(memory_space=pltpu.MemorySpace.HBM)],
        out_specs=pl.BlockSpec(memory_space=pltpu.MemorySpace.HBM),
        out_shape=jax.ShapeDtypeStruct((4, t_static, d), pos_weight.dtype),
        scratch_shapes=[
            pltpu.VMEM((NBUF, T_BLK, d), jnp.float32),
            pltpu.SemaphoreType.DMA((NBUF,)),
            pltpu.SemaphoreType.DMA((NBUF,)),
        ],
    )(pos_weight)
    return out


# speedup vs baseline: 1.4375x; 1.0108x over previous
import jax
import jax.numpy as jnp
from jax import lax
from jax.experimental import pallas as pl
from jax.experimental.pallas import tpu as pltpu

T_BLK = 1024
NBUF = 4


def _dma_body(w_hbm, o_hbm, buf, rsem, wsem):
    n = w_hbm.shape[0] // T_BLK

    def rd(i, s):
        return pltpu.make_async_copy(
            w_hbm.at[pl.ds(i * T_BLK, T_BLK)], buf.at[s], rsem.at[s]
        )

    def wr(b, i, s):
        return pltpu.make_async_copy(
            buf.at[s], o_hbm.at[b, pl.ds(i * T_BLK, T_BLK)], wsem.at[s]
        )

    for s in range(NBUF):
        rd(s, s).start()

    def step(g, carry):
        for s in range(NBUF):
            i = g * NBUF + s
            rd(i, s).wait()
            for b in range(4):
                wr(b, i, s).start()
        for s in range(NBUF):
            i = g * NBUF + s
            for b in range(4):
                wr(b, i, s).wait()
            nxt = i + NBUF

            @pl.when(nxt < n)
            def _():
                rd(nxt, s).start()

        return carry

    lax.fori_loop(0, n // NBUF, step, 0)


def kernel(B, T, pos_weight):
    t_static, d = pos_weight.shape
    out = pl.pallas_call(
        _dma_body,
        in_specs=[pl.BlockSpec(memory_space=pltpu.MemorySpace.HBM)],
        out_specs=pl.BlockSpec(memory_space=pltpu.MemorySpace.HBM),
        out_shape=jax.ShapeDtypeStruct((4, t_static, d), pos_weight.dtype),
        scratch_shapes=[
            pltpu.VMEM((NBUF, T_BLK, d), jnp.float32),
            pltpu.SemaphoreType.DMA((NBUF,)),
            pltpu.SemaphoreType.DMA((NBUF,)),
        ],
    )(pos_weight)
    return out


# speedup vs baseline: 1.4391x; 1.0011x over previous
import jax
import jax.numpy as jnp
from jax import lax
from jax.experimental import pallas as pl
from jax.experimental.pallas import tpu as pltpu

T_BLK = 2048
NBUF = 2


def _dma_body(w_hbm, o_hbm, buf, rsem, wsem):
    n = w_hbm.shape[0] // T_BLK

    def rd(i, s):
        return pltpu.make_async_copy(
            w_hbm.at[pl.ds(i * T_BLK, T_BLK)], buf.at[s], rsem.at[s]
        )

    def wr(b, i, s):
        return pltpu.make_async_copy(
            buf.at[s], o_hbm.at[b, pl.ds(i * T_BLK, T_BLK)], wsem.at[s]
        )

    for s in range(NBUF):
        rd(s, s).start()

    def step(g, carry):
        for s in range(NBUF):
            i = g * NBUF + s
            rd(i, s).wait()
            for b in range(4):
                wr(b, i, s).start()
        for s in range(NBUF):
            i = g * NBUF + s
            for b in range(4):
                wr(b, i, s).wait()
            nxt = i + NBUF

            @pl.when(nxt < n)
            def _():
                rd(nxt, s).start()

        return carry

    lax.fori_loop(0, n // NBUF, step, 0)


def kernel(B, T, pos_weight):
    t_static, d = pos_weight.shape
    out = pl.pallas_call(
        _dma_body,
        in_specs=[pl.BlockSpec(memory_space=pltpu.MemorySpace.HBM)],
        out_specs=pl.BlockSpec(memory_space=pltpu.MemorySpace.HBM),
        out_shape=jax.ShapeDtypeStruct((4, t_static, d), pos_weight.dtype),
        scratch_shapes=[
            pltpu.VMEM((NBUF, T_BLK, d), jnp.float32),
            pltpu.SemaphoreType.DMA((NBUF,)),
            pltpu.SemaphoreType.DMA((NBUF,)),
        ],
    )(pos_weight)
    return out
